# Initial kernel scaffold; baseline (speedup 1.0000x reference)
import jax
import jax.numpy as jnp
from jax.experimental import pallas as pl


def kernel(positions, net_to_pin, pin_to_macro, pin_offsets, rotation_onehot, net_weights):
    raise NotImplementedError("write your pallas kernel here")


# speedup vs baseline: 12.7984x; 12.7984x over previous
import functools

import numpy as np
import jax
import jax.numpy as jnp
from jax import lax
from jax.experimental import pallas as pl
from jax.experimental.pallas import tpu as pltpu
from jax.experimental.pallas import tpu_sc as plsc

_GAMMA = 10.0
_GRID = 64
_THRESH = 1.0
_SIGMA = 1.5
_CONG_W = 0.1
_STEEP = 2.0

_B, _V, _P, _N, _MP = 4, 512, 8192, 2048, 16
_NW = 32
_WPB = _NW // _B
_NPW = _N // _WPB
_NGRP = _NPW // 16


def _gauss_band_matrix():
    ksize = max(int(4 * _SIGMA) | 1, 3)
    half = ksize // 2
    x = np.arange(ksize, dtype=np.float64) - half
    g1 = np.exp(-(x ** 2) / (2 * _SIGMA ** 2))
    g1 = g1 / g1.sum()
    A = np.zeros((_GRID, _GRID), dtype=np.float32)
    for i in range(_GRID):
        for j in range(max(0, i - half), min(_GRID, i + half + 1)):
            A[i, j] = g1[j - i + half]
    return A


_GAUSS_A = jnp.asarray(_gauss_band_matrix())


def _sc_stats_body(ntp_hbm, p2m_hbm, offx_hbm, offy_hbm, posx_hbm, posy_hbm,
                   oh_hbm, out_hbm,
                   ntp_v, p2m_v, offx_v, offy_v, posx_v, posy_v, oh_v, res_v):
    wid = lax.axis_index("s") * 2 + lax.axis_index("c")
    b = wid // _WPB
    n0 = (wid % _WPB) * _NPW

    pltpu.sync_copy(ntp_hbm.at[pl.ds(n0 * _MP, _NPW * _MP)], ntp_v)
    pltpu.sync_copy(p2m_hbm, p2m_v)
    pltpu.sync_copy(offx_hbm, offx_v)
    pltpu.sync_copy(offy_hbm, offy_v)
    pltpu.sync_copy(posx_hbm.at[pl.ds(b * _V, _V)], posx_v)
    pltpu.sync_copy(posy_hbm.at[pl.ds(b * _V, _V)], posy_v)
    pltpu.sync_copy(oh_hbm.at[pl.ds(b * _V * 4, _V * 4)], oh_v)

    lanes = lax.broadcasted_iota(jnp.int32, (16,), 0)
    zero = jnp.zeros((16,), jnp.float32)
    neg = jnp.full((16,), -1e9, jnp.float32)
    pos = jnp.full((16,), 1e9, jnp.float32)

    def group(t, carry):
        row = t * (16 * _MP) + lanes * _MP
        sgx, snx, sgy, sny = zero, zero, zero, zero
        bxmax, bymax = neg, neg
        bxmin, bymin = pos, pos
        for m in range(_MP):
            idx = plsc.load_gather(ntp_v, [row + m])
            valid = idx >= 0
            safe = jnp.maximum(idx, 0)
            mac = plsc.load_gather(p2m_v, [safe])
            px = plsc.load_gather(posx_v, [mac])
            py = plsc.load_gather(posy_v, [mac])
            mac4 = mac * 4
            o0 = plsc.load_gather(oh_v, [mac4])
            o1 = plsc.load_gather(oh_v, [mac4 + 1])
            o2 = plsc.load_gather(oh_v, [mac4 + 2])
            o3 = plsc.load_gather(oh_v, [mac4 + 3])
            ox = plsc.load_gather(offx_v, [safe])
            oy = plsc.load_gather(offy_v, [safe])
            c = o0 - o2
            s = o1 - o3
            x = px + c * ox - s * oy
            y = py + s * ox + c * oy
            sgx = sgx + jnp.where(valid, jnp.exp(_GAMMA * x), zero)
            snx = snx + jnp.where(valid, jnp.exp(-_GAMMA * x), zero)
            sgy = sgy + jnp.where(valid, jnp.exp(_GAMMA * y), zero)
            sny = sny + jnp.where(valid, jnp.exp(-_GAMMA * y), zero)
            bxmax = jnp.maximum(bxmax, jnp.where(valid, x, neg))
            bxmin = jnp.minimum(bxmin, jnp.where(valid, x, pos))
            bymax = jnp.maximum(bymax, jnp.where(valid, y, neg))
            bymin = jnp.minimum(bymin, jnp.where(valid, y, pos))
        sl = pl.ds(t * 16, 16)
        res_v[0, sl] = sgx
        res_v[1, sl] = snx
        res_v[2, sl] = sgy
        res_v[3, sl] = sny
        res_v[4, sl] = bxmax
        res_v[5, sl] = bxmin
        res_v[6, sl] = bymax
        res_v[7, sl] = bymin
        return carry

    lax.fori_loop(0, _NGRP, group, 0)

    for k in range(8):
        pltpu.sync_copy(res_v.at[k], out_hbm.at[b * 8 + k, pl.ds(n0, _NPW)])


@functools.lru_cache(maxsize=1)
def _sc_stats():
    return pl.kernel(
        _sc_stats_body,
        mesh=plsc.VectorSubcoreMesh(core_axis_name="c", subcore_axis_name="s"),
        compiler_params=pltpu.CompilerParams(needs_layout_passes=False),
        out_type=jax.ShapeDtypeStruct((8 * _B, _N), jnp.float32),
        scratch_types=[
            pltpu.VMEM((_NPW * _MP,), jnp.int32),
            pltpu.VMEM((_P,), jnp.int32),
            pltpu.VMEM((_P,), jnp.float32),
            pltpu.VMEM((_P,), jnp.float32),
            pltpu.VMEM((_V,), jnp.float32),
            pltpu.VMEM((_V,), jnp.float32),
            pltpu.VMEM((_V * 4,), jnp.float32),
            pltpu.VMEM((8, _NPW), jnp.float32),
        ],
    )


def _tc_tail_body(sums_ref, w_ref, A_ref, tot_ref, hpwl_ref, pen_ref):
    A = A_ref[...]
    w = w_ref[...]
    col = lax.broadcasted_iota(jnp.int32, (_GRID, _N), 0).astype(jnp.float32)

    def sig(z):
        return 1.0 / (1.0 + jnp.exp(-z))

    for b in range(_B):
        S = sums_ref[b * 8:(b + 1) * 8, :]
        logs = jnp.log(S[0:4, :])
        wl = jnp.sum(logs, axis=0, keepdims=True) * (1.0 / _GAMMA)
        hp = jnp.sum(wl * w)

        scale = 0.5 * (_GRID - 1)
        gxmax = (S[4:5, :] + 1.0) * scale
        gxmin = (S[5:6, :] + 1.0) * scale
        gymax = (S[6:7, :] + 1.0) * scale
        gymin = (S[7:8, :] + 1.0) * scale

        wx = sig(_STEEP * (col - gxmin)) * sig(_STEEP * (gxmax - col))
        wy = sig(_STEEP * (col - gymin)) * sig(_STEEP * (gymax - col))
        area = jnp.clip((gxmax - gxmin + 1.0) * (gymax - gymin + 1.0),
                        1.0, None)
        wxa = wx / area

        rudy = lax.dot_general(
            wy, wxa, (((1,), (1,)), ((), ())),
            preferred_element_type=jnp.float32,
            precision=lax.Precision.HIGHEST)
        sm = jnp.dot(A, rudy, preferred_element_type=jnp.float32,
                     precision=lax.Precision.HIGHEST)
        sm = jnp.dot(sm, A, preferred_element_type=jnp.float32,
                     precision=lax.Precision.HIGHEST)
        o = jnp.maximum(sm - _THRESH, 0.0)
        pen = jnp.sum(o * o)

        hpwl_ref[b] = hp
        pen_ref[b] = pen
        tot_ref[b] = hp + _CONG_W * pen


def _tc_tail(sums, weights_row, A):
    return pl.pallas_call(
        _tc_tail_body,
        out_shape=(
            jax.ShapeDtypeStruct((_B,), jnp.float32),
            jax.ShapeDtypeStruct((_B,), jnp.float32),
            jax.ShapeDtypeStruct((_B,), jnp.float32),
        ),
        in_specs=[
            pl.BlockSpec(memory_space=pltpu.VMEM),
            pl.BlockSpec(memory_space=pltpu.VMEM),
            pl.BlockSpec(memory_space=pltpu.VMEM),
        ],
        out_specs=(
            pl.BlockSpec(memory_space=pltpu.SMEM),
            pl.BlockSpec(memory_space=pltpu.SMEM),
            pl.BlockSpec(memory_space=pltpu.SMEM),
        ),
    )(sums, weights_row, A)


def kernel(positions, net_to_pin, pin_to_macro, pin_offsets, rotation_onehot,
           net_weights):
    ntp = net_to_pin.astype(jnp.int32).reshape(-1)
    p2m = pin_to_macro.astype(jnp.int32)
    offx = pin_offsets[:, 0].astype(jnp.float32)
    offy = pin_offsets[:, 1].astype(jnp.float32)
    posx = positions[:, :, 0].reshape(-1).astype(jnp.float32)
    posy = positions[:, :, 1].reshape(-1).astype(jnp.float32)
    ohf = rotation_onehot.astype(jnp.float32).reshape(-1)

    sums = _sc_stats()(ntp, p2m, offx, offy, posx, posy, ohf)

    total, hpwl, penalty = _tc_tail(sums, net_weights.reshape(1, _N),
                                    _GAUSS_A)
    return total, hpwl, penalty
